# baseline (speedup 1.0000x reference)
import functools

import jax
import jax.numpy as jnp
from jax import lax
from jax.experimental import pallas as pl
from jax.experimental.pallas import tpu as pltpu
from jax.experimental.pallas import tpu_sc as plsc

N_OUT = 4096
N_CMD = 4
BATCH = 4096

NC = 2
NS = 16
L = 16
NW = NC * NS
ROWS_PER_W = BATCH // NW
CR = 4
N_CHUNK = ROWS_PER_W // CR


def _sc_body(x_hbm, cmd_hbm, out_hbm, in_v, out_v, cmd_v):
    wid = lax.axis_index("s") * NC + lax.axis_index("c")
    base = wid * ROWS_PER_W

    pltpu.sync_copy(cmd_hbm, cmd_v)
    cvec = cmd_v[...]
    iota4 = lax.iota(jnp.int32, L) * N_CMD
    colbase = iota4 + cvec

    def row(i, carry):
        r = base + i
        pltpu.sync_copy(x_hbm.at[r], in_v)

        def jbody(j, c2):
            col = j * (N_CMD * L) + colbase
            vals = plsc.load_gather(in_v, [col])
            out_v[pl.ds(j * L, L)] = vals
            return c2

        lax.fori_loop(0, N_OUT // L, jbody, 0, unroll=8)
        pltpu.sync_copy(out_v, out_hbm.at[r])
        return carry

    lax.fori_loop(0, ROWS_PER_W, row, 0)


@jax.jit
def kernel(x, command):
    cmd16 = jnp.broadcast_to(command.astype(jnp.int32), (L,))
    mesh = plsc.VectorSubcoreMesh(core_axis_name="c", subcore_axis_name="s")
    run = pl.kernel(
        _sc_body,
        out_type=jax.ShapeDtypeStruct((BATCH, N_OUT), jnp.float32),
        mesh=mesh,
        scratch_types=[
            pltpu.VMEM((N_OUT * N_CMD,), jnp.float32),
            pltpu.VMEM((N_OUT,), jnp.float32),
            pltpu.VMEM((L,), jnp.int32),
        ],
        compiler_params=pltpu.CompilerParams(needs_layout_passes=False),
    )
    return run(x, cmd16)
